# baseline (speedup 1.0000x reference)
import jax
import jax.numpy as jnp
from jax import lax
from jax.experimental import pallas as pl
from jax.experimental.pallas import tpu as pltpu

S = 8
R = 64
SR = S * R
DIN = 2048
DOUT = 4096
T = 2048
BLK = 512
NBLK = T // BLK
HD = DOUT // 2


def _body(sid_ref, x_ref, a_ref, b_ref, o_ref, a16_ref, bt_ref, mi_ref):
    i = pl.program_id(0)
    j = pl.program_id(1)

    @pl.when(jnp.logical_and(i == 0, j == 0))
    def _():
        a16_ref[...] = a_ref[...].astype(jnp.bfloat16)
        for s in range(S):
            bt_ref[pl.ds(s * R, R), :] = (
                jnp.transpose(b_ref[s]).astype(jnp.bfloat16))

    @pl.when(j == 0)
    def _():
        xb = x_ref[...].astype(jnp.bfloat16)
        inter = lax.dot_general(xb, a16_ref[...], (((1,), (1,)), ((), ())),
                                preferred_element_type=jnp.float32)
        band = lax.broadcasted_iota(jnp.int32, (BLK, SR), 1) // R
        mask = band == sid_ref[...]
        mi_ref[...] = jnp.where(mask, inter, 0.0).astype(jnp.bfloat16)

    o_ref[...] = lax.dot_general(
        mi_ref[...], bt_ref[:, pl.ds(j * HD, HD)], (((1,), (0,)), ((), ())),
        preferred_element_type=jnp.float32)


def kernel(x, slot_ids, layer_idx, A, B):
    del layer_idx
    sid_col = slot_ids.astype(jnp.int32).reshape(T, 1)
    a_all = A.reshape(SR, DIN)
    return pl.pallas_call(
        _body,
        grid=(NBLK, 2),
        in_specs=[
            pl.BlockSpec((BLK, 1), lambda i, j: (i, 0)),
            pl.BlockSpec((BLK, DIN), lambda i, j: (i, 0)),
            pl.BlockSpec((SR, DIN), lambda i, j: (0, 0)),
            pl.BlockSpec((S, DOUT, R), lambda i, j: (0, 0, 0)),
        ],
        out_specs=pl.BlockSpec((BLK, HD), lambda i, j: (i, j)),
        out_shape=jax.ShapeDtypeStruct((T, DOUT), jnp.float32),
        scratch_shapes=[
            pltpu.VMEM((SR, DIN), jnp.bfloat16),
            pltpu.VMEM((SR, DOUT), jnp.bfloat16),
            pltpu.VMEM((BLK, SR), jnp.bfloat16),
        ],
    )(sid_col, x, a_all, B)


# speedup vs baseline: 1.2009x; 1.2009x over previous
import jax
import jax.numpy as jnp
from jax import lax
from jax.experimental import pallas as pl
from jax.experimental.pallas import tpu as pltpu

S = 8
R = 64
SR = S * R
DIN = 2048
DOUT = 4096
T = 2048
BLK = 512
NBLK = T // BLK


def _body(sid_ref, x_ref, a_ref, b_ref, o_ref, a16_ref, bt_ref):
    i = pl.program_id(0)

    @pl.when(i == 0)
    def _():
        for s in range(S):
            a16_ref[pl.ds(s * R, R), :] = a_ref[s].astype(jnp.bfloat16)
            bt_ref[pl.ds(s * R, R), :] = (
                jnp.transpose(b_ref[s]).astype(jnp.bfloat16))

    xb = x_ref[...].astype(jnp.bfloat16)
    inter = lax.dot_general(xb, a16_ref[...], (((1,), (1,)), ((), ())),
                            preferred_element_type=jnp.float32)
    band = lax.broadcasted_iota(jnp.int32, (BLK, SR), 1) // R
    mask = band == sid_ref[...]
    minter = jnp.where(mask, inter, 0.0).astype(jnp.bfloat16)
    o_ref[...] = lax.dot_general(minter, bt_ref[...], (((1,), (0,)), ((), ())),
                                 preferred_element_type=jnp.float32)


def kernel(x, slot_ids, layer_idx, A, B):
    del layer_idx
    sid_col = slot_ids.astype(jnp.int32).reshape(T, 1)
    return pl.pallas_call(
        _body,
        grid=(NBLK,),
        in_specs=[
            pl.BlockSpec((BLK, 1), lambda i: (i, 0)),
            pl.BlockSpec((BLK, DIN), lambda i: (i, 0)),
            pl.BlockSpec((S, R, DIN), lambda i: (0, 0, 0)),
            pl.BlockSpec((S, DOUT, R), lambda i: (0, 0, 0)),
        ],
        out_specs=pl.BlockSpec((BLK, DOUT), lambda i: (i, 0)),
        out_shape=jax.ShapeDtypeStruct((T, DOUT), jnp.float32),
        scratch_shapes=[
            pltpu.VMEM((SR, DIN), jnp.bfloat16),
            pltpu.VMEM((SR, DOUT), jnp.bfloat16),
        ],
    )(sid_col, x, A, B)
